# baseline (speedup 1.0000x reference)
import functools

import jax
import jax.numpy as jnp
from jax import lax
from jax.experimental import pallas as pl
from jax.experimental.pallas import tpu as pltpu
from jax.experimental.pallas import tpu_sc as plsc

F = 26
CARD = 1000000
C = 2
B = 16384

NC = 2
NS = 16
NW = NC * NS
L = 16

B_PER_W = B // NW
PER_W = B_PER_W * F
CHUNK = 128
N_CHUNKS0 = PER_W // CHUNK
VREGS_PER_CHUNK = CHUNK // L
N_GROUPS = B_PER_W // L
LOG2_BPW = 9


@functools.partial(
    pl.kernel,
    out_type=jax.ShapeDtypeStruct((B * C,), jnp.float32),
    mesh=plsc.VectorSubcoreMesh(core_axis_name="c", subcore_axis_name="s"),
    scratch_types=[
        pltpu.VMEM((PER_W,), jnp.int32),
        pltpu.VMEM((2 * N_CHUNKS0, CHUNK), jnp.int32),
        pltpu.VMEM((2 * PER_W,), jnp.float32),
        pltpu.VMEM((C * B_PER_W,), jnp.float32),
        pltpu.VMEM((C * L,), jnp.float32),
        pltpu.SemaphoreType.DMA,
        pltpu.SemaphoreType.DMA,
    ],
)
def _wide_linear(xt_hbm, table_hbm, bias_hbm, out_hbm,
                 xv, idx, rows, outv, bias_v, sem, gsem):
    wid = lax.axis_index("s") * NC + lax.axis_index("c")

    pltpu.sync_copy(bias_hbm, bias_v)
    def copy_ids(f, carry):
        pltpu.async_copy(xt_hbm.at[pl.ds(f * B + wid * B_PER_W, B_PER_W)],
                         xv.at[pl.ds(f * B_PER_W, B_PER_W)], gsem)
        return carry
    lax.fori_loop(0, F, copy_ids, 0)

    def drain_ids(f, carry):
        pltpu.make_async_copy(
            xt_hbm.at[pl.ds(f * B + wid * B_PER_W, B_PER_W)],
            xv.at[pl.ds(f * B_PER_W, B_PER_W)], gsem).wait()
        return carry
    lax.fori_loop(0, F, drain_ids, 0)

    iota = lax.iota(jnp.int32, L)

    def fire(j, carry):
        for k in range(VREGS_PER_CHUNK):
            off = j * CHUNK + k * L
            f = lax.shift_right_logical(off, LOG2_BPW)
            i0 = (xv[pl.ds(off, L)] + f * CARD) * 2
            idx[j, pl.ds(k * L, L)] = i0
            idx[N_CHUNKS0 + j, pl.ds(k * L, L)] = i0 + 1
        pltpu.async_copy(table_hbm.at[idx.at[j]],
                         rows.at[pl.ds(j * CHUNK, CHUNK)], sem)
        pltpu.async_copy(table_hbm.at[idx.at[N_CHUNKS0 + j]],
                         rows.at[pl.ds(PER_W + j * CHUNK, CHUNK)], sem)
        return carry

    lax.fori_loop(0, N_CHUNKS0, fire, 0)

    def drain(j, carry):
        pltpu.make_async_copy(table_hbm.at[idx.at[j]],
                              rows.at[pl.ds(j * CHUNK, CHUNK)], sem).wait()
        pltpu.make_async_copy(table_hbm.at[idx.at[N_CHUNKS0 + j]],
                              rows.at[pl.ds(PER_W + j * CHUNK, CHUNK)],
                              sem).wait()
        return carry

    lax.fori_loop(0, N_CHUNKS0, drain, 0)

    b0 = bias_v[pl.ds(0, L)]
    b1 = bias_v[pl.ds(L, L)]

    def group(g, carry):
        base = g * L
        acc0 = b0
        acc1 = b1
        for f in range(F):
            acc0 = acc0 + rows[pl.ds(f * B_PER_W + base, L)]
            acc1 = acc1 + rows[pl.ds(PER_W + f * B_PER_W + base, L)]
        m = jnp.maximum(acc0, acc1)
        e0 = jnp.exp(acc0 - m)
        e1 = jnp.exp(acc1 - m)
        s = e0 + e1
        outv[pl.ds(base, L)] = e0 / s
        outv[pl.ds(B_PER_W + base, L)] = e1 / s
        return carry

    lax.fori_loop(0, N_GROUPS, group, 0)

    pltpu.sync_copy(outv.at[pl.ds(0, B_PER_W)],
                    out_hbm.at[pl.ds(wid * B_PER_W, B_PER_W)])
    pltpu.sync_copy(outv.at[pl.ds(B_PER_W, B_PER_W)],
                    out_hbm.at[pl.ds(B + wid * B_PER_W, B_PER_W)])


def kernel(x_ids, W, bias):
    table = W.reshape(F * CARD * C)
    xt = x_ids.T.reshape(-1)
    biasb = jnp.broadcast_to(bias[:, None], (C, L)).reshape(-1)
    out = _wide_linear(xt, table, biasb)
    return out.reshape(C, B).T


# speedup vs baseline: 8.2492x; 8.2492x over previous
import functools

import jax
import jax.numpy as jnp
from jax import lax
from jax.experimental import pallas as pl
from jax.experimental.pallas import tpu as pltpu
from jax.experimental.pallas import tpu_sc as plsc

F = 26
CARD = 1000000
C = 2
B = 16384

NC = 2
NS = 16
NW = NC * NS
L = 16

B_PER_W = B // NW
PER_W = B_PER_W * F
CHUNK = 128
N_CHUNKS0 = PER_W // CHUNK
VREGS_PER_CHUNK = CHUNK // L
N_GROUPS = B_PER_W // L
LOG2_BPW = 9


@functools.partial(
    pl.kernel,
    out_type=jax.ShapeDtypeStruct((B * C,), jnp.float32),
    mesh=plsc.VectorSubcoreMesh(core_axis_name="c", subcore_axis_name="s"),
    scratch_types=[
        pltpu.VMEM((PER_W,), jnp.int32),
        pltpu.VMEM((2 * N_CHUNKS0, CHUNK), jnp.int32),
        pltpu.VMEM((2 * PER_W,), jnp.float32),
        pltpu.VMEM((C * B_PER_W,), jnp.float32),
        pltpu.VMEM((C * L,), jnp.float32),
        pltpu.SemaphoreType.DMA,
        pltpu.SemaphoreType.DMA,
    ],
)
def _wide_linear(xt_hbm, table_hbm, bias_hbm, out_hbm,
                 xv, idx, rows, outv, bias_v, sem, gsem):
    wid = lax.axis_index("s") * NC + lax.axis_index("c")

    pltpu.sync_copy(bias_hbm, bias_v)
    def copy_ids(f, carry):
        pltpu.async_copy(xt_hbm.at[pl.ds(f * B + wid * B_PER_W, B_PER_W)],
                         xv.at[pl.ds(f * B_PER_W, B_PER_W)], gsem)
        return carry
    lax.fori_loop(0, F, copy_ids, 0)

    def drain_ids(f, carry):
        pltpu.make_async_copy(
            xt_hbm.at[pl.ds(f * B + wid * B_PER_W, B_PER_W)],
            xv.at[pl.ds(f * B_PER_W, B_PER_W)], gsem).wait()
        return carry
    lax.fori_loop(0, F, drain_ids, 0)

    iota = lax.iota(jnp.int32, L)

    def fire(j, carry):
        for k in range(VREGS_PER_CHUNK):
            off = j * CHUNK + k * L
            f = lax.shift_right_logical(off, LOG2_BPW)
            i0 = xv[pl.ds(off, L)] + f * (2 * CARD)
            idx[j, pl.ds(k * L, L)] = i0
            idx[N_CHUNKS0 + j, pl.ds(k * L, L)] = i0 + CARD
        pltpu.async_copy(table_hbm.at[idx.at[j]],
                         rows.at[pl.ds(j * CHUNK, CHUNK)], sem)
        pltpu.async_copy(table_hbm.at[idx.at[N_CHUNKS0 + j]],
                         rows.at[pl.ds(PER_W + j * CHUNK, CHUNK)], sem)
        return carry

    lax.fori_loop(0, N_CHUNKS0, fire, 0)

    def drain(j, carry):
        pltpu.make_async_copy(table_hbm.at[idx.at[j]],
                              rows.at[pl.ds(j * CHUNK, CHUNK)], sem).wait()
        pltpu.make_async_copy(table_hbm.at[idx.at[N_CHUNKS0 + j]],
                              rows.at[pl.ds(PER_W + j * CHUNK, CHUNK)],
                              sem).wait()
        return carry

    lax.fori_loop(0, N_CHUNKS0, drain, 0)

    b0 = bias_v[pl.ds(0, L)]
    b1 = bias_v[pl.ds(L, L)]

    def group(g, carry):
        base = g * L
        acc0 = b0
        acc1 = b1
        for f in range(F):
            acc0 = acc0 + rows[pl.ds(f * B_PER_W + base, L)]
            acc1 = acc1 + rows[pl.ds(PER_W + f * B_PER_W + base, L)]
        m = jnp.maximum(acc0, acc1)
        e0 = jnp.exp(acc0 - m)
        e1 = jnp.exp(acc1 - m)
        s = e0 + e1
        outv[pl.ds(base, L)] = e0 / s
        outv[pl.ds(B_PER_W + base, L)] = e1 / s
        return carry

    lax.fori_loop(0, N_GROUPS, group, 0)

    pltpu.sync_copy(outv.at[pl.ds(0, B_PER_W)],
                    out_hbm.at[pl.ds(wid * B_PER_W, B_PER_W)])
    pltpu.sync_copy(outv.at[pl.ds(B_PER_W, B_PER_W)],
                    out_hbm.at[pl.ds(B + wid * B_PER_W, B_PER_W)])


def kernel(x_ids, W, bias):
    table = W.transpose(0, 2, 1).reshape(F * CARD * C)
    xt = x_ids.T.reshape(-1)
    biasb = jnp.broadcast_to(bias[:, None], (C, L)).reshape(-1)
    out = _wide_linear(xt, table, biasb)
    return out.reshape(C, B).T


# speedup vs baseline: 64.4617x; 7.8143x over previous
import functools

import jax
import jax.numpy as jnp
from jax import lax
from jax.experimental import pallas as pl
from jax.experimental.pallas import tpu as pltpu
from jax.experimental.pallas import tpu_sc as plsc

F = 26
CARD = 1000000
C = 2
B = 16384

NC = 2
NS = 16
NW = NC * NS
L = 16

B_PER_W = B // NW
PER_W = B_PER_W * F
CHUNK = 128
N_CHUNKS0 = PER_W // CHUNK
VREGS_PER_CHUNK = CHUNK // L
N_GROUPS = B_PER_W // L
LOG2_BPW = 9

MAXID = 999936
PLANE = 999936
FSTRIDE = 2 * PLANE
N_TAIL = CARD - MAXID


@functools.partial(
    pl.kernel,
    out_type=jax.ShapeDtypeStruct((B * C,), jnp.float32),
    mesh=plsc.VectorSubcoreMesh(core_axis_name="c", subcore_axis_name="s"),
    scratch_types=[
        pltpu.VMEM((PER_W,), jnp.int32),
        pltpu.VMEM((2 * N_CHUNKS0, CHUNK), jnp.int32),
        pltpu.VMEM((2 * PER_W,), jnp.float32),
        pltpu.VMEM((C * B_PER_W,), jnp.float32),
        pltpu.VMEM((C * L,), jnp.float32),
        pltpu.VMEM((F * C * N_TAIL + L,), jnp.float32),
        pltpu.SemaphoreType.DMA,
        pltpu.SemaphoreType.DMA,
    ],
)
def _wide_linear(xt_hbm, table_hbm, tail_hbm, bias_hbm, out_hbm,
                 xv, idx, rows, outv, bias_v, tail_v, sem, gsem):
    wid = lax.axis_index("s") * NC + lax.axis_index("c")

    pltpu.sync_copy(bias_hbm, bias_v)
    pltpu.async_copy(tail_hbm, tail_v.at[pl.ds(0, F * C * N_TAIL)], gsem)
    def copy_ids(f, carry):
        pltpu.async_copy(xt_hbm.at[pl.ds(f * B + wid * B_PER_W, B_PER_W)],
                         xv.at[pl.ds(f * B_PER_W, B_PER_W)], gsem)
        return carry
    lax.fori_loop(0, F, copy_ids, 0)

    pltpu.make_async_copy(tail_hbm, tail_v.at[pl.ds(0, F * C * N_TAIL)], gsem).wait()
    def drain_ids(f, carry):
        pltpu.make_async_copy(
            xt_hbm.at[pl.ds(f * B + wid * B_PER_W, B_PER_W)],
            xv.at[pl.ds(f * B_PER_W, B_PER_W)], gsem).wait()
        return carry
    lax.fori_loop(0, F, drain_ids, 0)

    def fire(j, carry):
        for k in range(VREGS_PER_CHUNK):
            off = j * CHUNK + k * L
            f = lax.shift_right_logical(off, LOG2_BPW)
            ids = jnp.minimum(xv[pl.ds(off, L)], MAXID - 1)
            ib = lax.shift_right_logical(ids, 7)
            il = lax.bitwise_and(ids, 127)
            i0 = f * FSTRIDE + ib * 128 + il
            idx[j, pl.ds(k * L, L)] = i0
            idx[N_CHUNKS0 + j, pl.ds(k * L, L)] = i0 + PLANE
        pltpu.async_copy(table_hbm.at[idx.at[j]],
                         rows.at[pl.ds(j * CHUNK, CHUNK)], sem)
        pltpu.async_copy(table_hbm.at[idx.at[N_CHUNKS0 + j]],
                         rows.at[pl.ds(PER_W + j * CHUNK, CHUNK)], sem)
        return carry

    lax.fori_loop(0, N_CHUNKS0, fire, 0)

    def drain(j, carry):
        pltpu.make_async_copy(table_hbm.at[idx.at[j]],
                              rows.at[pl.ds(j * CHUNK, CHUNK)], sem).wait()
        pltpu.make_async_copy(table_hbm.at[idx.at[N_CHUNKS0 + j]],
                              rows.at[pl.ds(PER_W + j * CHUNK, CHUNK)],
                              sem).wait()
        return carry

    lax.fori_loop(0, N_CHUNKS0, drain, 0)

    iota = lax.iota(jnp.int32, L)

    def patch(j, carry):
        base = j * CHUNK
        m = xv[pl.ds(base, L)]
        for k in range(1, VREGS_PER_CHUNK):
            m = jnp.maximum(m, xv[pl.ds(base + k * L, L)])
        s = m[0]
        for l in range(1, L):
            s = jnp.maximum(s, m[l])

        @pl.when(s >= MAXID)
        def _():
            f = lax.shift_right_logical(base, LOG2_BPW)
            tbase = f * (2 * N_TAIL)
            for k in range(VREGS_PER_CHUNK):
                off = base + k * L
                ids = xv[pl.ds(off, L)]
                r0 = rows[pl.ds(off, L)]
                r1 = rows[pl.ds(PER_W + off, L)]
                for l in range(L):
                    idl = ids[l]
                    dl = jnp.maximum(idl - MAXID, 0)
                    w0 = tail_v[pl.ds(tbase + dl, L)]
                    w1 = tail_v[pl.ds(tbase + N_TAIL + dl, L)]
                    gate = jnp.where(idl >= MAXID, 1, 0)
                    onehot = (1 - jnp.minimum(jnp.abs(iota - l), 1)) * gate
                    cf = onehot.astype(jnp.float32)
                    r0 = r0 * (1.0 - cf) + w0[0] * cf
                    r1 = r1 * (1.0 - cf) + w1[0] * cf
                rows[pl.ds(off, L)] = r0
                rows[pl.ds(PER_W + off, L)] = r1
        return carry

    lax.fori_loop(0, N_CHUNKS0, patch, 0)

    b0 = bias_v[pl.ds(0, L)]
    b1 = bias_v[pl.ds(L, L)]

    def group(g, carry):
        base = g * L
        acc0 = b0
        acc1 = b1
        for f in range(F):
            acc0 = acc0 + rows[pl.ds(f * B_PER_W + base, L)]
            acc1 = acc1 + rows[pl.ds(PER_W + f * B_PER_W + base, L)]
        m = jnp.maximum(acc0, acc1)
        e0 = jnp.exp(acc0 - m)
        e1 = jnp.exp(acc1 - m)
        s = e0 + e1
        outv[pl.ds(base, L)] = e0 / s
        outv[pl.ds(B_PER_W + base, L)] = e1 / s
        return carry

    lax.fori_loop(0, N_GROUPS, group, 0)

    pltpu.sync_copy(outv.at[pl.ds(0, B_PER_W)],
                    out_hbm.at[pl.ds(wid * B_PER_W, B_PER_W)])
    pltpu.sync_copy(outv.at[pl.ds(B_PER_W, B_PER_W)],
                    out_hbm.at[pl.ds(B + wid * B_PER_W, B_PER_W)])


def kernel(x_ids, W, bias):
    Wt = W.transpose(0, 2, 1)
    Wm = Wt[:, :, :MAXID]
    t2 = jax.lax.optimization_barrier(Wm.reshape(F * C * 7812, 128))
    table = t2.reshape(-1)
    tail = Wt[:, :, MAXID:].reshape(-1)
    xt = x_ids.T.reshape(-1)
    biasb = jnp.broadcast_to(bias[:, None], (C, L)).reshape(-1)
    out = _wide_linear(xt, table, tail, biasb)
    return out.reshape(C, B).T
